# baseline (speedup 1.0000x reference)
import jax
import jax.numpy as jnp
from jax.experimental import pallas as pl
from jax.experimental.pallas import tpu as pltpu

_K = 256
_ROWS = 64
_COLS = 8192
_BLOCK_ROWS = 8


def _sparsify_kernel(x_ref, o_ref):
    x = x_ref[...]
    i = jax.lax.bitcast_convert_type(x, jnp.int32)
    keys = jnp.where(i >= 0, i, i ^ jnp.int32(0x7FFFFFFF))
    int_min = jnp.int32(-2147483648)

    def body(j, tb):
        bit = jnp.int32(1) << (jnp.int32(31) - j)
        trial = tb | bit
        thresh = trial ^ int_min
        t = (keys >= thresh).astype(jnp.int32)
        w = _COLS
        while w > 128:
            w //= 2
            t = t[:, :w] + t[:, w:]
        cnt = jnp.sum(t, axis=1, keepdims=True)
        return jnp.where(cnt >= _K, trial, tb)

    tb0 = jnp.zeros((x.shape[0], 1), jnp.int32)
    tb = jax.lax.fori_loop(0, 32, body, tb0)
    tkey = tb ^ int_min
    o_ref[...] = jnp.where(keys >= tkey, x, jnp.float32(0.0))


def kernel(x):
    grid = (_ROWS // _BLOCK_ROWS,)
    return pl.pallas_call(
        _sparsify_kernel,
        grid=grid,
        in_specs=[pl.BlockSpec((_BLOCK_ROWS, _COLS), lambda i: (i, 0))],
        out_specs=pl.BlockSpec((_BLOCK_ROWS, _COLS), lambda i: (i, 0)),
        out_shape=jax.ShapeDtypeStruct((_ROWS, _COLS), jnp.float32),
    )(x)


# speedup vs baseline: 1.8689x; 1.8689x over previous
import jax
import jax.numpy as jnp
from jax.experimental import pallas as pl
from jax.experimental.pallas import tpu as pltpu

_K = 256
_ROWS = 64
_COLS = 8192
_BLOCK_ROWS = 32


def _sparsify_kernel(x_ref, o_ref):
    x = x_ref[...]
    i = jax.lax.bitcast_convert_type(x, jnp.int32)
    keys = jnp.where(i >= 0, i, i ^ jnp.int32(0x7FFFFFFF))
    int_min = jnp.int32(-2147483648)

    def body(j, tb):
        bit = jnp.int32(1) << (jnp.int32(31) - j)
        trial = tb | bit
        thresh = trial ^ int_min
        cnt = jnp.sum((keys >= thresh).astype(jnp.int32), axis=1,
                      keepdims=True)
        return jnp.where(cnt >= _K, trial, tb)

    tb0 = jnp.zeros((x.shape[0], 1), jnp.int32)
    tb = jax.lax.fori_loop(0, 32, body, tb0)
    tkey = tb ^ int_min
    o_ref[...] = jnp.where(keys >= tkey, x, jnp.float32(0.0))


def kernel(x):
    grid = (_ROWS // _BLOCK_ROWS,)
    return pl.pallas_call(
        _sparsify_kernel,
        grid=grid,
        in_specs=[pl.BlockSpec((_BLOCK_ROWS, _COLS), lambda i: (i, 0))],
        out_specs=pl.BlockSpec((_BLOCK_ROWS, _COLS), lambda i: (i, 0)),
        out_shape=jax.ShapeDtypeStruct((_ROWS, _COLS), jnp.float32),
    )(x)


# speedup vs baseline: 2.2739x; 1.2167x over previous
import jax
import jax.numpy as jnp
from jax.experimental import pallas as pl
from jax.experimental.pallas import tpu as pltpu

_K = 256
_ROWS = 64
_COLS = 8192
_BLOCK_ROWS = 64


def _sparsify_kernel(x_ref, o_ref):
    x = x_ref[...]
    i = jax.lax.bitcast_convert_type(x, jnp.int32)
    keys = jnp.where(i >= 0, i, i ^ jnp.int32(0x7FFFFFFF))
    int_min = jnp.int32(-2147483648)

    def body(j, tb):
        bit = jnp.int32(1) << (jnp.int32(31) - j)
        trial = tb | bit
        thresh = trial ^ int_min
        cnt = jnp.sum((keys >= thresh).astype(jnp.int32), axis=1,
                      keepdims=True)
        return jnp.where(cnt >= _K, trial, tb)

    tb0 = jnp.zeros((x.shape[0], 1), jnp.int32)
    tb = jax.lax.fori_loop(0, 32, body, tb0)
    tkey = tb ^ int_min
    o_ref[...] = jnp.where(keys >= tkey, x, jnp.float32(0.0))


def kernel(x):
    grid = (_ROWS // _BLOCK_ROWS,)
    return pl.pallas_call(
        _sparsify_kernel,
        grid=grid,
        in_specs=[pl.BlockSpec((_BLOCK_ROWS, _COLS), lambda i: (i, 0))],
        out_specs=pl.BlockSpec((_BLOCK_ROWS, _COLS), lambda i: (i, 0)),
        out_shape=jax.ShapeDtypeStruct((_ROWS, _COLS), jnp.float32),
    )(x)
